# baseline (speedup 1.0000x reference)
import numpy as np
import jax
import jax.numpy as jnp
from jax.experimental import pallas as pl
from jax.experimental.pallas import tpu as pltpu

N = 10000
NFEAT = 128
NHID = 64
NCLASS = 16

BR = 400
BC = 1280
NB = N // BR
NJ = -(-N // BC)
ZPAD = NJ * BC

_CP = pltpu.CompilerParams(
    vmem_limit_bytes=64 * 1024 * 1024,
    dimension_semantics=("arbitrary",),
)


def _build_schedule():
    role, adjidx, zk, di, dj = [], [], [], [], []
    for k in range(NB):
        role.append(1)
        adjidx.append(k)
        zk.append(k)
        di.append(-1)
        dj.append(-1)
        for J in range(NJ):
            rj = min(NB - 1, (min((J + 1) * BC, N) + BR - 1) // BR - 1)
            for i in range(NB):
                if max(i, rj) == k:
                    role.append(0)
                    adjidx.append(min(k + 1, NB - 1))
                    zk.append(k)
                    di.append(i)
                    dj.append(J)
    lastdi, lastdj = 0, 0
    for t in range(len(role) - 1, -1, -1):
        if role[t] == 0:
            lastdi, lastdj = di[t], dj[t]
        else:
            di[t], dj[t] = lastdi, lastdj
    arrs = [np.array(a, dtype=np.int32) for a in (role, adjidx, zk, di, dj)]
    assert len(role) == NB + NB * NJ
    return arrs


_ROLE, _ADJIDX, _ZK, _DI, _DJ = _build_schedule()
_T = len(_ROLE)


def _c1_kernel(x_ref, w1_ref, o_ref):
    o_ref[...] = jnp.dot(x_ref[...], w1_ref[...],
                         preferred_element_type=jnp.float32)


def _hidden_kernel(adj_ref, c1_ref, b1_ref, o_ref):
    acc = jnp.dot(adj_ref[...], c1_ref[...],
                  preferred_element_type=jnp.float32)
    o_ref[...] = jax.nn.relu(acc + b1_ref[...])


def _fused_kernel(role_ref, adjidx_ref, zk_ref, di_ref, dj_ref,
                  adj_ref, h_ref, wc_ref, bc_ref, eps_ref,
                  o_ref, z_scr):
    t = pl.program_id(0)

    @pl.when(role_ref[t] == 1)
    def _z_step():
        k = zk_ref[t]
        h2 = jnp.dot(adj_ref[...], h_ref[...],
                     preferred_element_type=jnp.float32)
        p = jnp.dot(h2, wc_ref[...], preferred_element_type=jnp.float32)
        p = p + bc_ref[...]
        mu = p[:, :NCLASS]
        ls = p[:, NCLASS:]
        z_scr[pl.ds(k * BR, BR), :] = eps_ref[...] * jnp.exp(ls) + mu

    @pl.when(role_ref[t] == 0)
    def _d_step():
        i = di_ref[t]
        J = dj_ref[t]
        zi = z_scr[pl.ds(i * BR, BR), :]
        zj = z_scr[pl.ds(J * BC, BC), :]
        logits = jax.lax.dot_general(
            zi, zj, dimension_numbers=(((1,), (1,)), ((), ())),
            preferred_element_type=jnp.float32)
        o_ref[...] = jax.nn.sigmoid(logits)


def kernel(x, adj, W1, b1, W_mu, b_mu, W_ls, b_ls, eps):
    c1 = pl.pallas_call(
        _c1_kernel,
        out_shape=jax.ShapeDtypeStruct((N, NHID), jnp.float32),
    )(x, W1)

    hidden = pl.pallas_call(
        _hidden_kernel,
        grid=(NB,),
        in_specs=[
            pl.BlockSpec((BR, N), lambda i: (i, 0)),
            pl.BlockSpec((N, NHID), lambda i: (0, 0)),
            pl.BlockSpec((1, NHID), lambda i: (0, 0)),
        ],
        out_specs=pl.BlockSpec((BR, NHID), lambda i: (i, 0)),
        out_shape=jax.ShapeDtypeStruct((N, NHID), jnp.float32),
        compiler_params=_CP,
    )(adj, c1, b1.reshape(1, NHID))

    wc = jnp.concatenate([W_mu, W_ls], axis=1)
    bc = jnp.concatenate([b_mu, b_ls]).reshape(1, 2 * NCLASS)

    a_pred = pl.pallas_call(
        _fused_kernel,
        grid_spec=pltpu.PrefetchScalarGridSpec(
            num_scalar_prefetch=5,
            grid=(_T,),
            in_specs=[
                pl.BlockSpec((BR, N), lambda t, ro, ai, zk, di, dj: (ai[t], 0)),
                pl.BlockSpec((N, NHID), lambda t, ro, ai, zk, di, dj: (0, 0)),
                pl.BlockSpec((NHID, 2 * NCLASS),
                             lambda t, ro, ai, zk, di, dj: (0, 0)),
                pl.BlockSpec((1, 2 * NCLASS),
                             lambda t, ro, ai, zk, di, dj: (0, 0)),
                pl.BlockSpec((BR, NCLASS),
                             lambda t, ro, ai, zk, di, dj: (ai[t], 0)),
            ],
            out_specs=pl.BlockSpec(
                (BR, BC), lambda t, ro, ai, zk, di, dj: (di[t], dj[t])),
            scratch_shapes=[pltpu.VMEM((ZPAD, NCLASS), jnp.float32)],
        ),
        out_shape=jax.ShapeDtypeStruct((N, N), jnp.float32),
        compiler_params=_CP,
    )(jnp.asarray(_ROLE), jnp.asarray(_ADJIDX), jnp.asarray(_ZK),
      jnp.asarray(_DI), jnp.asarray(_DJ),
      adj, hidden, wc, bc, eps)

    return a_pred


# speedup vs baseline: 1.2143x; 1.2143x over previous
import jax
import jax.numpy as jnp
from jax.experimental import pallas as pl
from jax.experimental.pallas import tpu as pltpu

N = 10000
NFEAT = 128
NHID = 64
NCLASS = 16

BR = 400
_CP = pltpu.CompilerParams(
    vmem_limit_bytes=64 * 1024 * 1024,
    dimension_semantics=("parallel",),
)


def _c1_kernel(x_ref, w1_ref, o_ref):
    o_ref[...] = jnp.dot(x_ref[...], w1_ref[...],
                         preferred_element_type=jnp.float32)


def _hidden_kernel(adj_ref, c1_ref, b1_ref, o_ref):
    acc = jnp.dot(adj_ref[...], c1_ref[...],
                  preferred_element_type=jnp.float32)
    o_ref[...] = jax.nn.relu(acc + b1_ref[...])


def _z_kernel(adj_ref, h_ref, wc_ref, bc_ref, eps_ref, o_ref):
    h2 = jnp.dot(adj_ref[...], h_ref[...],
                 preferred_element_type=jnp.float32)
    p = jnp.dot(h2, wc_ref[...], preferred_element_type=jnp.float32)
    p = p + bc_ref[...]
    mu = p[:, :NCLASS]
    ls = p[:, NCLASS:]
    o_ref[...] = eps_ref[...] * jnp.exp(ls) + mu


def _decode_kernel(zr_ref, z_ref, o_ref):
    logits = jax.lax.dot_general(
        zr_ref[...], z_ref[...],
        dimension_numbers=(((1,), (1,)), ((), ())),
        preferred_element_type=jnp.float32)
    o_ref[...] = jax.nn.sigmoid(logits)


def kernel(x, adj, W1, b1, W_mu, b_mu, W_ls, b_ls, eps):
    nb = N // BR

    c1 = pl.pallas_call(
        _c1_kernel,
        out_shape=jax.ShapeDtypeStruct((N, NHID), jnp.float32),
    )(x, W1)

    hidden = pl.pallas_call(
        _hidden_kernel,
        grid=(nb,),
        in_specs=[
            pl.BlockSpec((BR, N), lambda i: (i, 0)),
            pl.BlockSpec((N, NHID), lambda i: (0, 0)),
            pl.BlockSpec((1, NHID), lambda i: (0, 0)),
        ],
        out_specs=pl.BlockSpec((BR, NHID), lambda i: (i, 0)),
        out_shape=jax.ShapeDtypeStruct((N, NHID), jnp.float32),
        compiler_params=_CP,
    )(adj, c1, b1.reshape(1, NHID))

    wc = jnp.concatenate([W_mu, W_ls], axis=1)
    bc = jnp.concatenate([b_mu, b_ls]).reshape(1, 2 * NCLASS)
    z = pl.pallas_call(
        _z_kernel,
        grid=(nb,),
        in_specs=[
            pl.BlockSpec((BR, N), lambda i: (i, 0)),
            pl.BlockSpec((N, NHID), lambda i: (0, 0)),
            pl.BlockSpec((NHID, 2 * NCLASS), lambda i: (0, 0)),
            pl.BlockSpec((1, 2 * NCLASS), lambda i: (0, 0)),
            pl.BlockSpec((BR, NCLASS), lambda i: (i, 0)),
        ],
        out_specs=pl.BlockSpec((BR, NCLASS), lambda i: (i, 0)),
        out_shape=jax.ShapeDtypeStruct((N, NCLASS), jnp.float32),
        compiler_params=_CP,
    )(adj, hidden, wc, bc, eps)

    a_pred = pl.pallas_call(
        _decode_kernel,
        grid=(nb,),
        in_specs=[
            pl.BlockSpec((BR, NCLASS), lambda i: (i, 0)),
            pl.BlockSpec((N, NCLASS), lambda i: (0, 0)),
        ],
        out_specs=pl.BlockSpec((BR, N), lambda i: (i, 0)),
        out_shape=jax.ShapeDtypeStruct((N, N), jnp.float32),
        compiler_params=_CP,
    )(z, z)

    return a_pred
